# Initial kernel scaffold; baseline (speedup 1.0000x reference)
import jax
import jax.numpy as jnp
from jax.experimental import pallas as pl


def kernel(x, w1, b1, g1, be1, m1, v1, w2, b2, g2, be2, m2, v2, fw1, fb1, fw2, fb2, fw3, fb3):
    raise NotImplementedError("write your pallas kernel here")


# speedup vs baseline: 1.7664x; 1.7664x over previous
import jax
import jax.numpy as jnp
from jax.experimental import pallas as pl
from jax.experimental.pallas import tpu as pltpu

_BB = 256


def _lenet_kernel(x_ref, a1_ref, b1_ref, a2_ref, b2_ref,
                  f1_ref, fb1_ref, f2_ref, fb2_ref, f3_ref, fb3_ref,
                  o_ref, h1_ref, h2_ref):
    f32 = jnp.float32
    BB = x_ref.shape[2]

    a1 = a1_ref[...]
    b1 = b1_ref[...].reshape(1, 1, 32, 1)
    for j in range(7):
        F = x_ref[4 * j:4 * j + 6].reshape(180, BB)
        z = jnp.dot(a1, F, preferred_element_type=f32)
        z = z.reshape(2, 2, 28, 32, BB)
        z = jnp.maximum(z[:, 0], z[:, 1])
        z = z.reshape(2, 14, 2, 32, BB)
        z = jnp.maximum(z[:, :, 0], z[:, :, 1])
        h1_ref[2 * j:2 * j + 2] = jnp.maximum(z + b1, 0.0)

    b2 = b2_ref[...].reshape(1, 64, 1)
    for i in range(6):
        for t in (0, 1):
            zz = []
            for d in (0, 1):
                h = 2 * i + d
                acc = None
                for ky in range(3):
                    op = h1_ref[h + ky, 6 * t:6 * t + 8].reshape(256, BB)
                    p = jnp.dot(a2_ref[ky], op, preferred_element_type=f32)
                    acc = p if acc is None else acc + p
                zz.append(acc)
            m = jnp.maximum(zz[0], zz[1]).reshape(3, 2, 64, BB)
            m = jnp.maximum(m[:, 0], m[:, 1])
            h2_ref[i, 3 * t:3 * t + 3] = jnp.maximum(m + b2, 0.0)

    f = h2_ref[...].reshape(2304, BB)
    z = jnp.dot(f1_ref[...], f, preferred_element_type=f32) + fb1_ref[...]
    z = jnp.dot(f2_ref[...], z, preferred_element_type=f32) + fb2_ref[...]
    z = jnp.dot(f3_ref[...], z, preferred_element_type=f32) + fb3_ref[...]
    o_ref[...] = z


def kernel(x, w1, b1, g1, be1, m1, v1, w2, b2, g2, be2, m2, v2,
           fw1, fb1, fw2, fb2, fw3, fb3):
    eps = 1e-5
    B = x.shape[0]
    BB = _BB
    nb = pl.cdiv(B, BB)
    b_pad = nb * BB

    xt = x[:, 0].astype(jnp.float32)
    xt = jnp.pad(xt, ((0, b_pad - B), (1, 1), (1, 1)))
    xt = jnp.transpose(xt, (1, 2, 0))

    s1 = g1 / jnp.sqrt(v1 + eps)
    w1f = w1[:, 0] * s1[:, None, None]
    b1f = ((b1 - m1) * s1 + be1).reshape(32, 1)
    s2 = g2 / jnp.sqrt(v2 + eps)
    w2f = w2 * s2[:, None, None, None]
    b2f = ((b2 - m2) * s2 + be2).reshape(64, 1)

    r = jnp.arange(4)[:, None, None, None, None]
    w = jnp.arange(28)[None, :, None, None, None]
    c = jnp.arange(32)[None, None, :, None, None]
    ky6 = jnp.arange(6)[None, None, None, :, None]
    u = jnp.arange(30)[None, None, None, None, :]
    ky = ky6 - r
    kx = u - w
    ok = (ky >= 0) & (ky < 3) & (kx >= 0) & (kx < 3)
    a1 = jnp.where(ok, w1f[c, ky.clip(0, 2), kx.clip(0, 2)], 0.0)
    a1 = a1.reshape(3584, 180)

    kyA = jnp.arange(3)[:, None, None, None, None]
    w2l = jnp.arange(6)[None, :, None, None, None]
    c2 = jnp.arange(64)[None, None, :, None, None]
    ul = jnp.arange(8)[None, None, None, :, None]
    c1 = jnp.arange(32)[None, None, None, None, :]
    kx2 = ul - w2l
    ok2 = (kx2 >= 0) & (kx2 < 3)
    a2 = jnp.where(ok2, w2f[c2, c1, kyA, kx2.clip(0, 2)], 0.0)
    a2 = a2.reshape(3, 384, 256)

    f1t = (fw1.reshape(64, 6, 6, 600).transpose(3, 1, 2, 0)
           .reshape(600, 2304).astype(jnp.float32))
    f2t = fw2.T.astype(jnp.float32)
    f3t = fw3.T.astype(jnp.float32)
    fb1c = fb1.reshape(600, 1).astype(jnp.float32)
    fb2c = fb2.reshape(120, 1).astype(jnp.float32)
    fb3c = fb3.reshape(10, 1).astype(jnp.float32)

    c2d = lambda g: (0, 0)
    c3d = lambda g: (0, 0, 0)
    out = pl.pallas_call(
        _lenet_kernel,
        out_shape=jax.ShapeDtypeStruct((10, b_pad), jnp.float32),
        grid=(nb,),
        in_specs=[
            pl.BlockSpec((30, 30, BB), lambda g: (0, 0, g)),
            pl.BlockSpec((3584, 180), c2d),
            pl.BlockSpec((32, 1), c2d),
            pl.BlockSpec((3, 384, 256), c3d),
            pl.BlockSpec((64, 1), c2d),
            pl.BlockSpec((600, 2304), c2d),
            pl.BlockSpec((600, 1), c2d),
            pl.BlockSpec((120, 600), c2d),
            pl.BlockSpec((120, 1), c2d),
            pl.BlockSpec((10, 120), c2d),
            pl.BlockSpec((10, 1), c2d),
        ],
        out_specs=pl.BlockSpec((10, BB), lambda g: (0, g)),
        scratch_shapes=[
            pltpu.VMEM((14, 14, 32, BB), jnp.float32),
            pltpu.VMEM((6, 6, 64, BB), jnp.float32),
        ],
        compiler_params=pltpu.CompilerParams(
            dimension_semantics=("parallel",),
            vmem_limit_bytes=64 * 1024 * 1024),
    )(xt, a1, b1f, a2, b2f, f1t, fb1c, f2t, fb2c, f3t, fb3c)
    return out.T[:B]


# speedup vs baseline: 1.8155x; 1.0278x over previous
import jax
import jax.numpy as jnp
from jax.experimental import pallas as pl
from jax.experimental.pallas import tpu as pltpu

_BB = 256


def _lenet_kernel(x_ref, a1_ref, b1_ref, a2_ref, b2_ref,
                  f1_ref, fb1_ref, f2_ref, fb2_ref, f3_ref, fb3_ref,
                  o_ref, xs_ref, h1_ref, h2_ref):
    f32 = jnp.float32
    BB = x_ref.shape[0]

    xt = jnp.transpose(x_ref[...], (1, 0))
    xs_ref[0:1] = jnp.zeros((1, 28, BB), f32)
    xs_ref[29:30] = jnp.zeros((1, 28, BB), f32)
    xs_ref[1:29] = xt.reshape(28, 28, BB)

    a1 = a1_ref[...]
    b1 = b1_ref[...].reshape(1, 1, 32, 1)
    for j in range(7):
        F = xs_ref[4 * j:4 * j + 6].reshape(168, BB)
        z = jnp.dot(a1, F, preferred_element_type=f32)
        z = z.reshape(2, 2, 28, 32, BB)
        z = jnp.maximum(z[:, 0], z[:, 1])
        z = z.reshape(2, 14, 2, 32, BB)
        z = jnp.maximum(z[:, :, 0], z[:, :, 1])
        h1_ref[2 * j:2 * j + 2] = jnp.maximum(z + b1, 0.0)

    b2 = b2_ref[...].reshape(1, 64, 1)
    for i in range(6):
        for t in (0, 1):
            zz = []
            for d in (0, 1):
                h = 2 * i + d
                acc = None
                for ky in range(3):
                    op = h1_ref[h + ky, 6 * t:6 * t + 8].reshape(256, BB)
                    p = jnp.dot(a2_ref[ky], op, preferred_element_type=f32)
                    acc = p if acc is None else acc + p
                zz.append(acc)
            m = jnp.maximum(zz[0], zz[1]).reshape(3, 2, 64, BB)
            m = jnp.maximum(m[:, 0], m[:, 1])
            h2_ref[i, 3 * t:3 * t + 3] = jnp.maximum(m + b2, 0.0)

    f = h2_ref[...].reshape(2304, BB)
    z = jnp.dot(f1_ref[...], f, preferred_element_type=f32) + fb1_ref[...]
    z = jnp.dot(f2_ref[...], z, preferred_element_type=f32) + fb2_ref[...]
    z = jnp.dot(f3_ref[...], z, preferred_element_type=f32) + fb3_ref[...]
    o_ref[...] = jnp.transpose(z, (1, 0))


def kernel(x, w1, b1, g1, be1, m1, v1, w2, b2, g2, be2, m2, v2,
           fw1, fb1, fw2, fb2, fw3, fb3):
    eps = 1e-5
    B = x.shape[0]
    BB = _BB
    nb = pl.cdiv(B, BB)
    b_pad = nb * BB

    xt = x.reshape(B, 784).astype(jnp.float32)
    xt = jnp.pad(xt, ((0, b_pad - B), (0, 0)))

    s1 = g1 / jnp.sqrt(v1 + eps)
    w1f = w1[:, 0] * s1[:, None, None]
    b1f = ((b1 - m1) * s1 + be1).reshape(32, 1)
    s2 = g2 / jnp.sqrt(v2 + eps)
    w2f = w2 * s2[:, None, None, None]
    b2f = ((b2 - m2) * s2 + be2).reshape(64, 1)

    r = jnp.arange(4)[:, None, None, None, None]
    w = jnp.arange(28)[None, :, None, None, None]
    c = jnp.arange(32)[None, None, :, None, None]
    ky6 = jnp.arange(6)[None, None, None, :, None]
    u = jnp.arange(28)[None, None, None, None, :]
    ky = ky6 - r
    kx = u - w + 1
    ok = (ky >= 0) & (ky < 3) & (kx >= 0) & (kx < 3)
    a1 = jnp.where(ok, w1f[c, ky.clip(0, 2), kx.clip(0, 2)], 0.0)
    a1 = a1.reshape(3584, 168)

    kyA = jnp.arange(3)[:, None, None, None, None]
    w2l = jnp.arange(6)[None, :, None, None, None]
    c2 = jnp.arange(64)[None, None, :, None, None]
    ul = jnp.arange(8)[None, None, None, :, None]
    c1 = jnp.arange(32)[None, None, None, None, :]
    kx2 = ul - w2l
    ok2 = (kx2 >= 0) & (kx2 < 3)
    a2 = jnp.where(ok2, w2f[c2, c1, kyA, kx2.clip(0, 2)], 0.0)
    a2 = a2.reshape(3, 384, 256)

    f1t = (fw1.reshape(64, 6, 6, 600).transpose(3, 1, 2, 0)
           .reshape(600, 2304).astype(jnp.float32))
    f2t = fw2.T.astype(jnp.float32)
    f3t = fw3.T.astype(jnp.float32)
    fb1c = fb1.reshape(600, 1).astype(jnp.float32)
    fb2c = fb2.reshape(120, 1).astype(jnp.float32)
    fb3c = fb3.reshape(10, 1).astype(jnp.float32)

    c2d = lambda g: (0, 0)
    c3d = lambda g: (0, 0, 0)
    out = pl.pallas_call(
        _lenet_kernel,
        out_shape=jax.ShapeDtypeStruct((b_pad, 10), jnp.float32),
        grid=(nb,),
        in_specs=[
            pl.BlockSpec((BB, 784), lambda g: (g, 0)),
            pl.BlockSpec((3584, 168), c2d),
            pl.BlockSpec((32, 1), c2d),
            pl.BlockSpec((3, 384, 256), c3d),
            pl.BlockSpec((64, 1), c2d),
            pl.BlockSpec((600, 2304), c2d),
            pl.BlockSpec((600, 1), c2d),
            pl.BlockSpec((120, 600), c2d),
            pl.BlockSpec((120, 1), c2d),
            pl.BlockSpec((10, 120), c2d),
            pl.BlockSpec((10, 1), c2d),
        ],
        out_specs=pl.BlockSpec((BB, 10), lambda g: (g, 0)),
        scratch_shapes=[
            pltpu.VMEM((30, 28, BB), jnp.float32),
            pltpu.VMEM((14, 14, 32, BB), jnp.float32),
            pltpu.VMEM((6, 6, 64, BB), jnp.float32),
        ],
        compiler_params=pltpu.CompilerParams(
            dimension_semantics=("parallel",),
            vmem_limit_bytes=64 * 1024 * 1024),
    )(xt, a1, b1f, a2, b2f, f1t, fb1c, f2t, fb2c, f3t, fb3c)
    return out[:B]


# speedup vs baseline: 23.5381x; 12.9654x over previous
import numpy as np

import jax
import jax.numpy as jnp
from jax.experimental import pallas as pl
from jax.experimental.pallas import tpu as pltpu

_BB = 256


def _band_patterns():
    r = np.arange(4)[:, None, None, None]
    w = np.arange(28)[None, :, None, None]
    ky6 = np.arange(6)[None, None, :, None]
    u = np.arange(28)[None, None, None, :]
    m1 = np.zeros((9, 4, 28, 6, 28), np.float32)
    for ky in range(3):
        for kx in range(3):
            m1[ky * 3 + kx] = (ky6 == r + ky) & (u == w + kx - 1)
    w2l = np.arange(6)[:, None]
    ul = np.arange(8)[None, :]
    m2 = np.zeros((3, 6, 8), np.float32)
    for kx in range(3):
        m2[kx] = (ul == w2l + kx)
    return m1, m2


_M1, _M2 = _band_patterns()


def _lenet_kernel(x_ref, a1_ref, b1_ref, a2_ref, b2_ref,
                  f1_ref, fb1_ref, f2_ref, fb2_ref, f3_ref, fb3_ref,
                  o_ref, xs_ref, h1_ref, h2_ref):
    f32 = jnp.float32
    BB = x_ref.shape[0]

    xt = jnp.transpose(x_ref[...], (1, 0))
    xs_ref[0:1] = jnp.zeros((1, 28, BB), f32)
    xs_ref[29:30] = jnp.zeros((1, 28, BB), f32)
    xs_ref[1:29] = xt.reshape(28, 28, BB)

    a1 = a1_ref[...]
    b1 = b1_ref[...].reshape(1, 1, 32, 1)
    for j in range(7):
        F = xs_ref[4 * j:4 * j + 6].reshape(168, BB)
        z = jnp.dot(a1, F, preferred_element_type=f32)
        z = z.reshape(2, 2, 28, 32, BB)
        z = jnp.maximum(z[:, 0], z[:, 1])
        z = z.reshape(2, 14, 2, 32, BB)
        z = jnp.maximum(z[:, :, 0], z[:, :, 1])
        h1_ref[2 * j:2 * j + 2] = jnp.maximum(z + b1, 0.0)

    b2 = b2_ref[...].reshape(1, 64, 1)
    for i in range(6):
        for t in (0, 1):
            zz = []
            for d in (0, 1):
                h = 2 * i + d
                acc = None
                for ky in range(3):
                    op = h1_ref[h + ky, 6 * t:6 * t + 8].reshape(256, BB)
                    p = jnp.dot(a2_ref[ky], op, preferred_element_type=f32)
                    acc = p if acc is None else acc + p
                zz.append(acc)
            m = jnp.maximum(zz[0], zz[1]).reshape(3, 2, 64, BB)
            m = jnp.maximum(m[:, 0], m[:, 1])
            h2_ref[i, 3 * t:3 * t + 3] = jnp.maximum(m + b2, 0.0)

    f = h2_ref[...].reshape(2304, BB)
    z = jnp.dot(f1_ref[...], f, preferred_element_type=f32) + fb1_ref[...]
    z = jnp.dot(f2_ref[...], z, preferred_element_type=f32) + fb2_ref[...]
    z = jnp.dot(f3_ref[...], z, preferred_element_type=f32) + fb3_ref[...]
    o_ref[...] = jnp.transpose(z, (1, 0))


def kernel(x, w1, b1, g1, be1, m1, v1, w2, b2, g2, be2, m2, v2,
           fw1, fb1, fw2, fb2, fw3, fb3):
    eps = 1e-5
    B = x.shape[0]
    BB = _BB
    nb = pl.cdiv(B, BB)
    b_pad = nb * BB

    xt = x.reshape(B, 784).astype(jnp.float32)
    xt = jnp.pad(xt, ((0, b_pad - B), (0, 0)))

    s1 = g1 / jnp.sqrt(v1 + eps)
    w1f = w1[:, 0] * s1[:, None, None]
    b1f = ((b1 - m1) * s1 + be1).reshape(32, 1)
    s2 = g2 / jnp.sqrt(v2 + eps)
    w2f = w2 * s2[:, None, None, None]
    b2f = ((b2 - m2) * s2 + be2).reshape(64, 1)

    a1 = jnp.tensordot(w1f.reshape(32, 9), jnp.asarray(_M1), axes=[[1], [0]])
    a1 = a1.transpose(1, 2, 0, 3, 4).reshape(3584, 168)

    a2 = jnp.tensordot(w2f, jnp.asarray(_M2), axes=[[3], [0]])
    a2 = a2.transpose(2, 3, 0, 4, 1).reshape(3, 384, 256)

    f1t = (fw1.reshape(64, 6, 6, 600).transpose(3, 1, 2, 0)
           .reshape(600, 2304).astype(jnp.float32))
    f2t = fw2.T.astype(jnp.float32)
    f3t = fw3.T.astype(jnp.float32)
    fb1c = fb1.reshape(600, 1).astype(jnp.float32)
    fb2c = fb2.reshape(120, 1).astype(jnp.float32)
    fb3c = fb3.reshape(10, 1).astype(jnp.float32)

    c2d = lambda g: (0, 0)
    c3d = lambda g: (0, 0, 0)
    out = pl.pallas_call(
        _lenet_kernel,
        out_shape=jax.ShapeDtypeStruct((b_pad, 10), jnp.float32),
        grid=(nb,),
        in_specs=[
            pl.BlockSpec((BB, 784), lambda g: (g, 0)),
            pl.BlockSpec((3584, 168), c2d),
            pl.BlockSpec((32, 1), c2d),
            pl.BlockSpec((3, 384, 256), c3d),
            pl.BlockSpec((64, 1), c2d),
            pl.BlockSpec((600, 2304), c2d),
            pl.BlockSpec((600, 1), c2d),
            pl.BlockSpec((120, 600), c2d),
            pl.BlockSpec((120, 1), c2d),
            pl.BlockSpec((10, 120), c2d),
            pl.BlockSpec((10, 1), c2d),
        ],
        out_specs=pl.BlockSpec((BB, 10), lambda g: (g, 0)),
        scratch_shapes=[
            pltpu.VMEM((30, 28, BB), jnp.float32),
            pltpu.VMEM((14, 14, 32, BB), jnp.float32),
            pltpu.VMEM((6, 6, 64, BB), jnp.float32),
        ],
        compiler_params=pltpu.CompilerParams(
            dimension_semantics=("parallel",),
            vmem_limit_bytes=64 * 1024 * 1024),
    )(xt, a1, b1f, a2, b2f, f1t, fb1c, f2t, fb2c, f3t, fb3c)
    return out[:B]


# speedup vs baseline: 24.7642x; 1.0521x over previous
import numpy as np

import jax
import jax.numpy as jnp
from jax.experimental import pallas as pl
from jax.experimental.pallas import tpu as pltpu

_BB = 256


def _band_patterns():
    r = np.arange(4)[:, None, None, None]
    w = np.arange(28)[None, :, None, None]
    ky6 = np.arange(6)[None, None, :, None]
    u = np.arange(28)[None, None, None, :]
    m1 = np.zeros((9, 4, 28, 6, 28), np.float32)
    for ky in range(3):
        for kx in range(3):
            m1[ky * 3 + kx] = (ky6 == r + ky) & (u == w + kx - 1)
    w2l = np.arange(6)[:, None]
    ul = np.arange(8)[None, :]
    m2 = np.zeros((3, 6, 8), np.float32)
    for kx in range(3):
        m2[kx] = (ul == w2l + kx)
    return m1, m2


_M1, _M2 = _band_patterns()


def _lenet_kernel(x_ref, a1_ref, b1_ref, a2_ref, b2_ref,
                  f1_ref, fb1_ref, f2_ref, fb2_ref, f3_ref, fb3_ref,
                  o_ref, xs_ref, h1_ref, h2_ref):
    f32 = jnp.float32
    BB = x_ref.shape[0]

    bf16 = jnp.bfloat16

    xt = jnp.transpose(x_ref[...], (1, 0))
    xs_ref[0:1] = jnp.zeros((1, 28, BB), bf16)
    xs_ref[29:30] = jnp.zeros((1, 28, BB), bf16)
    xs_ref[1:29] = xt.reshape(28, 28, BB)

    a1 = a1_ref[...]
    b1 = b1_ref[...].reshape(1, 1, 32, 1)
    for j in range(7):
        F = xs_ref[4 * j:4 * j + 6].reshape(168, BB)
        z = jnp.dot(a1, F, preferred_element_type=f32)
        z = z.reshape(2, 2, 28, 32, BB)
        z = jnp.maximum(z[:, 0], z[:, 1])
        z = z.reshape(2, 14, 2, 32, BB)
        z = jnp.maximum(z[:, :, 0], z[:, :, 1])
        h1_ref[2 * j:2 * j + 2] = jnp.maximum(z + b1, 0.0).astype(bf16)

    b2 = b2_ref[...].reshape(1, 64, 1)
    for i in range(6):
        for t in (0, 1):
            zz = []
            for d in (0, 1):
                h = 2 * i + d
                acc = None
                for ky in range(3):
                    op = h1_ref[h + ky, 6 * t:6 * t + 8].reshape(256, BB)
                    p = jnp.dot(a2_ref[ky], op, preferred_element_type=f32)
                    acc = p if acc is None else acc + p
                zz.append(acc)
            m = jnp.maximum(zz[0], zz[1]).reshape(3, 2, 64, BB)
            m = jnp.maximum(m[:, 0], m[:, 1])
            h2_ref[i, 3 * t:3 * t + 3] = jnp.maximum(m + b2, 0.0).astype(bf16)

    f = h2_ref[...].reshape(2304, BB)
    z = jnp.dot(f1_ref[...], f, preferred_element_type=f32) + fb1_ref[...]
    z = jnp.dot(f2_ref[...], z.astype(bf16), preferred_element_type=f32) + fb2_ref[...]
    z = jnp.dot(f3_ref[...], z.astype(bf16), preferred_element_type=f32) + fb3_ref[...]
    o_ref[...] = jnp.transpose(z, (1, 0))


def kernel(x, w1, b1, g1, be1, m1, v1, w2, b2, g2, be2, m2, v2,
           fw1, fb1, fw2, fb2, fw3, fb3):
    eps = 1e-5
    B = x.shape[0]
    BB = _BB
    nb = pl.cdiv(B, BB)
    b_pad = nb * BB

    xt = x.reshape(B, 784).astype(jnp.bfloat16)
    xt = jnp.pad(xt, ((0, b_pad - B), (0, 0)))

    s1 = g1 / jnp.sqrt(v1 + eps)
    w1f = w1[:, 0] * s1[:, None, None]
    b1f = ((b1 - m1) * s1 + be1).reshape(32, 1)
    s2 = g2 / jnp.sqrt(v2 + eps)
    w2f = w2 * s2[:, None, None, None]
    b2f = ((b2 - m2) * s2 + be2).reshape(64, 1)

    a1 = jnp.tensordot(w1f.reshape(32, 9), jnp.asarray(_M1), axes=[[1], [0]])
    a1 = a1.transpose(1, 2, 0, 3, 4).reshape(3584, 168)
    a1 = a1.astype(jnp.bfloat16)

    a2 = jnp.tensordot(w2f, jnp.asarray(_M2), axes=[[3], [0]])
    a2 = a2.transpose(2, 3, 0, 4, 1).reshape(3, 384, 256)
    a2 = a2.astype(jnp.bfloat16)

    f1t = (fw1.reshape(64, 6, 6, 600).transpose(3, 1, 2, 0)
           .reshape(600, 2304).astype(jnp.bfloat16))
    f2t = fw2.T.astype(jnp.bfloat16)
    f3t = fw3.T.astype(jnp.bfloat16)
    fb1c = fb1.reshape(600, 1).astype(jnp.float32)
    fb2c = fb2.reshape(120, 1).astype(jnp.float32)
    fb3c = fb3.reshape(10, 1).astype(jnp.float32)

    c2d = lambda g: (0, 0)
    c3d = lambda g: (0, 0, 0)
    out = pl.pallas_call(
        _lenet_kernel,
        out_shape=jax.ShapeDtypeStruct((b_pad, 10), jnp.float32),
        grid=(nb,),
        in_specs=[
            pl.BlockSpec((BB, 784), lambda g: (g, 0)),
            pl.BlockSpec((3584, 168), c2d),
            pl.BlockSpec((32, 1), c2d),
            pl.BlockSpec((3, 384, 256), c3d),
            pl.BlockSpec((64, 1), c2d),
            pl.BlockSpec((600, 2304), c2d),
            pl.BlockSpec((600, 1), c2d),
            pl.BlockSpec((120, 600), c2d),
            pl.BlockSpec((120, 1), c2d),
            pl.BlockSpec((10, 120), c2d),
            pl.BlockSpec((10, 1), c2d),
        ],
        out_specs=pl.BlockSpec((BB, 10), lambda g: (g, 0)),
        scratch_shapes=[
            pltpu.VMEM((30, 28, BB), jnp.bfloat16),
            pltpu.VMEM((14, 14, 32, BB), jnp.bfloat16),
            pltpu.VMEM((6, 6, 64, BB), jnp.bfloat16),
        ],
        compiler_params=pltpu.CompilerParams(
            dimension_semantics=("parallel",),
            vmem_limit_bytes=64 * 1024 * 1024),
    )(xt, a1, b1f, a2, b2f, f1t, fb1c, f2t, fb2c, f3t, fb3c)
    return out[:B]
